# Initial kernel scaffold; baseline (speedup 1.0000x reference)
import jax
import jax.numpy as jnp
from jax.experimental import pallas as pl


def kernel(vectors, node_feats, node_specie, senders, receivers, W_skip, W_up, W_mlp1, W_mlp2, W_mlp3, W_down0, W_down1, W_down2):
    raise NotImplementedError("write your pallas kernel here")


# speedup vs baseline: 20.7125x; 20.7125x over previous
import functools

import jax
import jax.numpy as jnp
from jax import lax
from jax.experimental import pallas as pl
from jax.experimental.pallas import tpu as pltpu
from jax.experimental.pallas import tpu_sc as plsc

N_NODES = 10000
N_EDGES = 320000
D = 128
MUL = 32
NBASIS = 8
HID = 64
SCAL_OUT = 96
AVG_NEIGH = 32.0

NC = 2
NS = 16
CHUNK = 80
ROWS_2D = N_EDGES // CHUNK
G_CHUNKS = N_EDGES // (NC * NS) // CHUNK
S_CHUNKS = N_EDGES // NS // CHUNK
N_PAD = 10240
NODES_PER_TILE = N_PAD // NS
HALF = 176


def _swish(x):
    return x * (1.0 / (1.0 + jnp.exp(-x)))


def _node_kernel(nf_ref, sp_ref, wup_ref, wskip_ref, h_ref, self_ref):
    nf = nf_ref[...]
    sp = sp_ref[...]
    h_ref[...] = jnp.dot(nf, wup_ref[...], preferred_element_type=jnp.float32)
    acc = jnp.zeros((nf.shape[0], SCAL_OUT), jnp.float32)
    for s in range(5):
        mask = (sp == s).astype(jnp.float32)
        acc = acc + jnp.dot(nf * mask, wskip_ref[s],
                            preferred_element_type=jnp.float32)
    self_ref[...] = acc


def _gather_body(h_hbm, snd_hbm, m_hbm, idx_v, rows_v, sem):
    c = lax.axis_index("c")
    s = lax.axis_index("s")
    wid = s * NC + c

    def chunk(i, carry):
        r = wid * G_CHUNKS + i
        pltpu.sync_copy(snd_hbm.at[r], idx_v)
        pltpu.async_copy(h_hbm.at[idx_v], rows_v, sem).wait()
        pltpu.sync_copy(rows_v, m_hbm.at[pl.ds(r * CHUNK, CHUNK)])
        return carry

    lax.fori_loop(0, G_CHUNKS, chunk, 0)


def _edge_kernel(vec_ref, m_ref, w1_ref, w2_ref, w3_ref,
                 wd0_ref, wd1_ref, wd2_ref, ua_ref, ub_ref):
    v = vec_ref[...]
    x = v[:, 0:1]
    y = v[:, 1:2]
    z = v[:, 2:3]
    len2 = x * x + y * y + z * z
    length = jnp.sqrt(len2)
    safe = jnp.where(length == 0.0, 1.0, length)
    ux, uy, uz = x / safe, y / safe, z / safe

    s3 = 1.7320508075688772
    s15 = 3.872983346207417
    s5h = 1.118033988749895
    y1x, y1y, y1z = s3 * ux, s3 * uy, s3 * uz
    y2a = s15 * ux * uy
    y2b = s15 * uy * uz
    y2c = s5h * (3.0 * uz * uz - 1.0)
    y2d = s15 * ux * uz
    y2e = (s15 * 0.5) * (ux * ux - uy * uy)

    Be = v.shape[0]
    k = lax.broadcasted_iota(jnp.int32, (Be, NBASIS), 1).astype(jnp.float32) + 1.0
    inv_safe = 1.0 / safe
    bes = 1.4142135623730951 * jnp.sin(k * (jnp.pi * length)) * inv_safe
    l6 = len2 * len2 * len2
    l7 = l6 * length
    l8 = l7 * length
    env = 1.0 - 28.0 * l6 + 48.0 * l7 - 21.0 * l8
    env = jnp.where(length < 1.0, env, 0.0)
    rad = bes * env
    rad = jnp.where(length == 0.0, 0.0, rad)

    hm = _swish(jnp.dot(rad, w1_ref[...], preferred_element_type=jnp.float32))
    hm = _swish(jnp.dot(hm, w2_ref[...], preferred_element_type=jnp.float32))
    mix = jnp.dot(hm, w3_ref[...], preferred_element_type=jnp.float32)

    m = m_ref[...]
    q0 = jnp.dot(m * mix[:, 0:D], wd0_ref[...],
                 preferred_element_type=jnp.float32)
    q1 = jnp.dot(m * mix[:, D:2 * D], wd1_ref[...],
                 preferred_element_type=jnp.float32)
    q2 = jnp.dot(m * mix[:, 2 * D:3 * D], wd2_ref[...],
                 preferred_element_type=jnp.float32)

    u = jnp.concatenate(
        [q0, q1 * y1x, q1 * y1y, q1 * y1z,
         q2 * y2a, q2 * y2b, q2 * y2c, q2 * y2d, q2 * y2e], axis=1)
    ua_ref[...] = u[:, :HALF]
    ub_ref[...] = u[:, HALF:]


def _scatter_body(ua_hbm, ub_hbm, recv_hbm, z_hbm, outa_hbm, outb_hbm,
                  idx_v, u_v, acc, sem):
    c = lax.axis_index("c")
    sid = lax.axis_index("s")

    def run(u_hbm, out_hbm):
        pltpu.sync_copy(z_hbm, acc.at[pl.ds(sid * NODES_PER_TILE,
                                            NODES_PER_TILE)])
        plsc.subcore_barrier()

        def chunk(i, carry):
            r = sid * S_CHUNKS + i
            pltpu.sync_copy(recv_hbm.at[r], idx_v)
            pltpu.sync_copy(u_hbm.at[pl.ds(r * CHUNK, CHUNK)], u_v)
            pltpu.sync_copy(u_v, acc.at[idx_v], add=True)
            return carry

        lax.fori_loop(0, S_CHUNKS, chunk, 0)
        plsc.subcore_barrier()
        sl = pl.ds(sid * NODES_PER_TILE, NODES_PER_TILE)
        pltpu.sync_copy(acc.at[sl], out_hbm.at[sl])

    @pl.when(c == 0)
    def _():
        run(ua_hbm, outa_hbm)

    @pl.when(c == 1)
    def _():
        run(ub_hbm, outb_hbm)


def _out_kernel(aa_ref, ab_ref, self_ref, out_ref):
    inv = 1.0 / jnp.sqrt(AVG_NEIGH)
    aa = aa_ref[...]
    ab = ab_ref[...]
    s_pre = aa[:, :SCAL_OUT] * inv + self_ref[...]
    feat = s_pre[:, :MUL]
    gv = _swish(s_pre[:, MUL:2 * MUL])
    gt = _swish(s_pre[:, 2 * MUL:3 * MUL])
    acc_rest = jnp.concatenate([aa[:, SCAL_OUT:], ab], axis=1) * inv
    v_im = acc_rest[:, :96] * jnp.concatenate([gv, gv, gv], axis=1)
    t_im = acc_rest[:, 96:] * jnp.concatenate([gt, gt, gt, gt, gt], axis=1)
    out_ref[...] = jnp.concatenate([_swish(feat), v_im, t_im], axis=1)


def kernel(vectors, node_feats, node_specie, senders, receivers,
           W_skip, W_up, W_mlp1, W_mlp2, W_mlp3, W_down0, W_down1, W_down2):
    n = node_feats.shape[0]
    f32 = jnp.float32

    Bn = 1000
    h, self_conn = pl.pallas_call(
        _node_kernel,
        grid=(n // Bn,),
        in_specs=[
            pl.BlockSpec((Bn, D), lambda i: (i, 0)),
            pl.BlockSpec((Bn, 1), lambda i: (i, 0)),
            pl.BlockSpec((D, D), lambda i: (0, 0)),
            pl.BlockSpec((5, D, SCAL_OUT), lambda i: (0, 0, 0)),
        ],
        out_specs=[
            pl.BlockSpec((Bn, D), lambda i: (i, 0)),
            pl.BlockSpec((Bn, SCAL_OUT), lambda i: (i, 0)),
        ],
        out_shape=[
            jax.ShapeDtypeStruct((n, D), f32),
            jax.ShapeDtypeStruct((n, SCAL_OUT), f32),
        ],
    )(node_feats, node_specie.astype(jnp.int32).reshape(n, 1), W_up, W_skip)

    snd2d = senders.astype(jnp.int32).reshape(ROWS_2D, CHUNK)
    mesh = plsc.VectorSubcoreMesh(core_axis_name="c", subcore_axis_name="s")
    sc_params = pltpu.CompilerParams(use_tc_tiling_on_sc=False)
    m = pl.kernel(
        _gather_body,
        out_type=jax.ShapeDtypeStruct((N_EDGES, D), f32),
        mesh=mesh,
        compiler_params=sc_params,
        scratch_types=[
            pltpu.VMEM((CHUNK,), jnp.int32),
            pltpu.VMEM((CHUNK, D), f32),
            pltpu.SemaphoreType.DMA,
        ],
    )(h, snd2d)

    Be = 1000
    ua, ub = pl.pallas_call(
        _edge_kernel,
        grid=(N_EDGES // Be,),
        in_specs=[
            pl.BlockSpec((Be, 3), lambda i: (i, 0)),
            pl.BlockSpec((Be, D), lambda i: (i, 0)),
            pl.BlockSpec((NBASIS, HID), lambda i: (0, 0)),
            pl.BlockSpec((HID, HID), lambda i: (0, 0)),
            pl.BlockSpec((HID, 3 * D), lambda i: (0, 0)),
            pl.BlockSpec((D, SCAL_OUT), lambda i: (0, 0)),
            pl.BlockSpec((D, MUL), lambda i: (0, 0)),
            pl.BlockSpec((D, MUL), lambda i: (0, 0)),
        ],
        out_specs=[
            pl.BlockSpec((Be, HALF), lambda i: (i, 0)),
            pl.BlockSpec((Be, HALF), lambda i: (i, 0)),
        ],
        out_shape=[
            jax.ShapeDtypeStruct((N_EDGES, HALF), f32),
            jax.ShapeDtypeStruct((N_EDGES, HALF), f32),
        ],
    )(vectors, m, W_mlp1, W_mlp2, W_mlp3, W_down0, W_down1, W_down2)

    rcv2d = receivers.astype(jnp.int32).reshape(ROWS_2D, CHUNK)
    zrows = jnp.zeros((NODES_PER_TILE, HALF), f32)
    acc_a, acc_b = pl.kernel(
        _scatter_body,
        out_type=[
            jax.ShapeDtypeStruct((N_PAD, HALF), f32),
            jax.ShapeDtypeStruct((N_PAD, HALF), f32),
        ],
        mesh=mesh,
        compiler_params=sc_params,
        scratch_types=[
            pltpu.VMEM((CHUNK,), jnp.int32),
            pltpu.VMEM((CHUNK, HALF), f32),
            pltpu.VMEM_SHARED((N_PAD, HALF), f32),
            pltpu.SemaphoreType.DMA,
        ],
    )(ua, ub, rcv2d, zrows)

    out288 = pl.pallas_call(
        _out_kernel,
        grid=(n // Bn,),
        in_specs=[
            pl.BlockSpec((Bn, HALF), lambda i: (i, 0)),
            pl.BlockSpec((Bn, HALF), lambda i: (i, 0)),
            pl.BlockSpec((Bn, SCAL_OUT), lambda i: (i, 0)),
        ],
        out_specs=pl.BlockSpec((Bn, 288), lambda i: (i, 0)),
        out_shape=jax.ShapeDtypeStruct((n, 288), f32),
    )(acc_a, acc_b, self_conn)

    out_s = out288[:, :MUL]
    out_v = out288[:, MUL:MUL + 96].reshape(n, 3, MUL).transpose(0, 2, 1)
    out_t = out288[:, MUL + 96:].reshape(n, 5, MUL).transpose(0, 2, 1)
    return jnp.concatenate(
        [out_s, out_v.reshape(n, 96), out_t.reshape(n, 160)], axis=1)


# speedup vs baseline: 27.7788x; 1.3412x over previous
import functools

import jax
import jax.numpy as jnp
from jax import lax
from jax.experimental import pallas as pl
from jax.experimental.pallas import tpu as pltpu
from jax.experimental.pallas import tpu_sc as plsc

N_NODES = 10000
N_EDGES = 320000
D = 128
MUL = 32
NBASIS = 8
HID = 64
SCAL_OUT = 96
AVG_NEIGH = 32.0

NC = 2
NS = 16
CHUNK = 80
ROWS_2D = N_EDGES // CHUNK
G_CHUNKS = N_EDGES // (NC * NS) // CHUNK
S_CHUNKS = N_EDGES // NS // CHUNK
N_PAD = 10240
NODES_PER_TILE = N_PAD // NS
HALF = 176


def _swish(x):
    return x * (1.0 / (1.0 + jnp.exp(-x)))


def _node_kernel(nf_ref, sp_ref, wup_ref, wskip_ref, h_ref, self_ref):
    nf = nf_ref[...]
    sp = sp_ref[...]
    h_ref[...] = jnp.dot(nf, wup_ref[...], preferred_element_type=jnp.float32)
    acc = jnp.zeros((nf.shape[0], SCAL_OUT), jnp.float32)
    for s in range(5):
        mask = (sp == s).astype(jnp.float32)
        acc = acc + jnp.dot(nf * mask, wskip_ref[s],
                            preferred_element_type=jnp.float32)
    self_ref[...] = acc


def _gather_body(h_hbm, snd_hbm, m_hbm, idx_v, rows_v, sem):
    c = lax.axis_index("c")
    s = lax.axis_index("s")
    wid = s * NC + c

    def chunk(i, carry):
        r = wid * G_CHUNKS + i
        pltpu.sync_copy(snd_hbm.at[r], idx_v)
        pltpu.async_copy(h_hbm.at[idx_v], rows_v, sem).wait()
        pltpu.sync_copy(rows_v, m_hbm.at[pl.ds(r * CHUNK, CHUNK)])
        return carry

    lax.fori_loop(0, G_CHUNKS, chunk, 0)


def _edge_kernel(vec_ref, m_ref, w1_ref, w2_ref, w3_ref,
                 wd0_ref, wd12_ref, s_ref, cy_ref, by_ref, ua_ref, ub_ref):
    v = vec_ref[...]
    x = v[:, 0:1]
    y = v[:, 1:2]
    z = v[:, 2:3]
    len2 = x * x + y * y + z * z
    zmask = len2 == 0.0
    inv_safe = jnp.where(zmask, 1.0, lax.rsqrt(len2))
    length = len2 * inv_safe

    u3 = v * inv_safe
    u3r = jnp.concatenate([u3[:, 1:3], u3[:, 0:1]], axis=1)
    p9 = jnp.concatenate([u3, u3 * u3, u3 * u3r], axis=1)
    yb = jnp.dot(p9, cy_ref[...],
                 preferred_element_type=jnp.float32) + by_ref[...]

    Be = v.shape[0]
    k = lax.broadcasted_iota(jnp.int32, (Be, NBASIS), 1).astype(jnp.float32) + 1.0
    kx = k * length
    n_i = (kx + 0.5).astype(jnp.int32)
    r = (kx - n_i.astype(jnp.float32)) * jnp.pi
    r2 = r * r
    poly = r * (1.0 + r2 * (-0.16666667 + r2 * (8.3333333e-3
                + r2 * (-1.98412698e-4 + r2 * 2.75573192e-6))))
    sgn = jnp.where((n_i & 1) == 0, 1.0, -1.0)
    bes = (1.4142135623730951 * sgn * poly) * inv_safe
    l6 = len2 * len2 * len2
    l7 = l6 * length
    l8 = l7 * length
    env = 1.0 - 28.0 * l6 + 48.0 * l7 - 21.0 * l8
    env = jnp.where(length < 1.0, env, 0.0)
    rad = jnp.where(zmask, 0.0, bes * env)

    hm = _swish(jnp.dot(rad, w1_ref[...], preferred_element_type=jnp.float32))
    hm = _swish(jnp.dot(hm, w2_ref[...], preferred_element_type=jnp.float32))
    mix = jnp.dot(hm, w3_ref[...], preferred_element_type=jnp.float32)

    m = m_ref[...]
    q0 = jnp.dot(m * mix[:, 0:D], wd0_ref[...],
                 preferred_element_type=jnp.float32)
    a12 = jnp.concatenate([m * mix[:, D:2 * D], m * mix[:, 2 * D:3 * D]],
                          axis=1)
    q12 = jnp.dot(a12, wd12_ref[...],
                  preferred_element_type=jnp.float32)
    qq = jnp.dot(q12, s_ref[...],
                 preferred_element_type=jnp.float32)
    u = jnp.concatenate([q0, qq * yb], axis=1)
    ua_ref[...] = u[:, :HALF]
    ub_ref[...] = u[:, HALF:]


def _scatter_body(ua_hbm, ub_hbm, recv_hbm, z_hbm, outa_hbm, outb_hbm,
                  idx_v, u_v, acc, sem):
    c = lax.axis_index("c")
    sid = lax.axis_index("s")

    def run(u_hbm, out_hbm):
        pltpu.sync_copy(z_hbm, acc.at[pl.ds(sid * NODES_PER_TILE,
                                            NODES_PER_TILE)])
        plsc.subcore_barrier()

        def chunk(i, carry):
            r = sid * S_CHUNKS + i
            pltpu.sync_copy(recv_hbm.at[r], idx_v)
            pltpu.sync_copy(u_hbm.at[pl.ds(r * CHUNK, CHUNK)], u_v)
            pltpu.sync_copy(u_v, acc.at[idx_v], add=True)
            return carry

        lax.fori_loop(0, S_CHUNKS, chunk, 0)
        plsc.subcore_barrier()
        sl = pl.ds(sid * NODES_PER_TILE, NODES_PER_TILE)
        pltpu.sync_copy(acc.at[sl], out_hbm.at[sl])

    @pl.when(c == 0)
    def _():
        run(ua_hbm, outa_hbm)

    @pl.when(c == 1)
    def _():
        run(ub_hbm, outb_hbm)


def _out_kernel(aa_ref, ab_ref, self_ref, out_ref):
    inv = 1.0 / jnp.sqrt(AVG_NEIGH)
    aa = aa_ref[...]
    ab = ab_ref[...]
    s_pre = aa[:, :SCAL_OUT] * inv + self_ref[...]
    feat = s_pre[:, :MUL]
    gv = _swish(s_pre[:, MUL:2 * MUL])
    gt = _swish(s_pre[:, 2 * MUL:3 * MUL])
    acc_rest = jnp.concatenate([aa[:, SCAL_OUT:], ab], axis=1) * inv
    v_im = acc_rest[:, :96] * jnp.concatenate([gv, gv, gv], axis=1)
    t_im = acc_rest[:, 96:] * jnp.concatenate([gt, gt, gt, gt, gt], axis=1)
    out_ref[...] = jnp.concatenate([_swish(feat), v_im, t_im], axis=1)


def kernel(vectors, node_feats, node_specie, senders, receivers,
           W_skip, W_up, W_mlp1, W_mlp2, W_mlp3, W_down0, W_down1, W_down2):
    n = node_feats.shape[0]
    f32 = jnp.float32

    Bn = 1000
    h, self_conn = pl.pallas_call(
        _node_kernel,
        grid=(n // Bn,),
        in_specs=[
            pl.BlockSpec((Bn, D), lambda i: (i, 0)),
            pl.BlockSpec((Bn, 1), lambda i: (i, 0)),
            pl.BlockSpec((D, D), lambda i: (0, 0)),
            pl.BlockSpec((5, D, SCAL_OUT), lambda i: (0, 0, 0)),
        ],
        out_specs=[
            pl.BlockSpec((Bn, D), lambda i: (i, 0)),
            pl.BlockSpec((Bn, SCAL_OUT), lambda i: (i, 0)),
        ],
        out_shape=[
            jax.ShapeDtypeStruct((n, D), f32),
            jax.ShapeDtypeStruct((n, SCAL_OUT), f32),
        ],
    )(node_feats, node_specie.astype(jnp.int32).reshape(n, 1), W_up, W_skip)

    snd2d = senders.astype(jnp.int32).reshape(ROWS_2D, CHUNK)
    mesh = plsc.VectorSubcoreMesh(core_axis_name="c", subcore_axis_name="s")
    sc_params = pltpu.CompilerParams(use_tc_tiling_on_sc=False)
    m = pl.kernel(
        _gather_body,
        out_type=jax.ShapeDtypeStruct((N_EDGES, D), f32),
        mesh=mesh,
        compiler_params=sc_params,
        scratch_types=[
            pltpu.VMEM((CHUNK,), jnp.int32),
            pltpu.VMEM((CHUNK, D), f32),
            pltpu.SemaphoreType.DMA,
        ],
    )(h, snd2d)

    s3 = 1.7320508075688772
    s15 = 3.872983346207417
    s5h = 1.118033988749895
    C9 = jnp.zeros((9, 8), f32)
    C9 = C9.at[0, 0].set(s3).at[1, 1].set(s3).at[2, 2].set(s3)
    C9 = C9.at[6, 3].set(s15)
    C9 = C9.at[7, 4].set(s15)
    C9 = C9.at[5, 5].set(3.0 * s5h)
    C9 = C9.at[8, 6].set(s15)
    C9 = C9.at[3, 7].set(0.5 * s15).at[4, 7].set(-0.5 * s15)
    b8 = jnp.zeros((1, 8), f32).at[0, 5].set(-s5h)
    R = jnp.kron(jnp.eye(8, dtype=f32), jnp.ones((1, MUL), f32))
    CY = C9 @ R
    bY = b8 @ R
    I32 = jnp.eye(MUL, dtype=f32)
    S = jnp.concatenate([
        jnp.concatenate([jnp.tile(I32, (1, 3)), jnp.zeros((MUL, 160), f32)], 1),
        jnp.concatenate([jnp.zeros((MUL, 96), f32), jnp.tile(I32, (1, 5))], 1),
    ], axis=0)
    Wd12 = jnp.concatenate([
        jnp.concatenate([W_down1, jnp.zeros((D, MUL), f32)], 1),
        jnp.concatenate([jnp.zeros((D, MUL), f32), W_down2], 1),
    ], axis=0)

    Be = 2000
    ua, ub = pl.pallas_call(
        _edge_kernel,
        grid=(N_EDGES // Be,),
        in_specs=[
            pl.BlockSpec((Be, 3), lambda i: (i, 0)),
            pl.BlockSpec((Be, D), lambda i: (i, 0)),
            pl.BlockSpec((NBASIS, HID), lambda i: (0, 0)),
            pl.BlockSpec((HID, HID), lambda i: (0, 0)),
            pl.BlockSpec((HID, 3 * D), lambda i: (0, 0)),
            pl.BlockSpec((D, SCAL_OUT), lambda i: (0, 0)),
            pl.BlockSpec((2 * D, HID), lambda i: (0, 0)),
            pl.BlockSpec((HID, 2 * D), lambda i: (0, 0)),
            pl.BlockSpec((9, 2 * D), lambda i: (0, 0)),
            pl.BlockSpec((1, 2 * D), lambda i: (0, 0)),
        ],
        out_specs=[
            pl.BlockSpec((Be, HALF), lambda i: (i, 0)),
            pl.BlockSpec((Be, HALF), lambda i: (i, 0)),
        ],
        out_shape=[
            jax.ShapeDtypeStruct((N_EDGES, HALF), f32),
            jax.ShapeDtypeStruct((N_EDGES, HALF), f32),
        ],
    )(vectors, m, W_mlp1, W_mlp2, W_mlp3, W_down0, Wd12, S, CY, bY)

    rcv2d = receivers.astype(jnp.int32).reshape(ROWS_2D, CHUNK)
    zrows = jnp.zeros((NODES_PER_TILE, HALF), f32)
    acc_a, acc_b = pl.kernel(
        _scatter_body,
        out_type=[
            jax.ShapeDtypeStruct((N_PAD, HALF), f32),
            jax.ShapeDtypeStruct((N_PAD, HALF), f32),
        ],
        mesh=mesh,
        compiler_params=sc_params,
        scratch_types=[
            pltpu.VMEM((CHUNK,), jnp.int32),
            pltpu.VMEM((CHUNK, HALF), f32),
            pltpu.VMEM_SHARED((N_PAD, HALF), f32),
            pltpu.SemaphoreType.DMA,
        ],
    )(ua, ub, rcv2d, zrows)

    out288 = pl.pallas_call(
        _out_kernel,
        grid=(n // Bn,),
        in_specs=[
            pl.BlockSpec((Bn, HALF), lambda i: (i, 0)),
            pl.BlockSpec((Bn, HALF), lambda i: (i, 0)),
            pl.BlockSpec((Bn, SCAL_OUT), lambda i: (i, 0)),
        ],
        out_specs=pl.BlockSpec((Bn, 288), lambda i: (i, 0)),
        out_shape=jax.ShapeDtypeStruct((n, 288), f32),
    )(acc_a, acc_b, self_conn)

    out_s = out288[:, :MUL]
    out_v = out288[:, MUL:MUL + 96].reshape(n, 3, MUL).transpose(0, 2, 1)
    out_t = out288[:, MUL + 96:].reshape(n, 5, MUL).transpose(0, 2, 1)
    return jnp.concatenate(
        [out_s, out_v.reshape(n, 96), out_t.reshape(n, 160)], axis=1)


# speedup vs baseline: 31.6928x; 1.1409x over previous
import functools

import jax
import jax.numpy as jnp
from jax import lax
from jax.experimental import pallas as pl
from jax.experimental.pallas import tpu as pltpu
from jax.experimental.pallas import tpu_sc as plsc

N_NODES = 10000
N_EDGES = 320000
D = 128
MUL = 32
NBASIS = 8
HID = 64
SCAL_OUT = 96
AVG_NEIGH = 32.0

NC = 2
NS = 16
CHUNK = 80
ROWS_2D = N_EDGES // CHUNK
G_CHUNKS = N_EDGES // (NC * NS) // CHUNK
CHUNK_S = 40
ROWS_S = N_EDGES // CHUNK_S
S_CHUNKS = N_EDGES // NS // CHUNK_S
N_PAD = 10240
NODES_PER_TILE = N_PAD // NS
HALF = 176


def _swish(x):
    return x * (1.0 / (1.0 + jnp.exp(-x)))


def _node_kernel(nf_ref, sp_ref, wup_ref, wskip_ref, h_ref, self_ref):
    nf = nf_ref[...]
    sp = sp_ref[...]
    h_ref[...] = jnp.dot(nf, wup_ref[...], preferred_element_type=jnp.float32)
    acc = jnp.zeros((nf.shape[0], SCAL_OUT), jnp.float32)
    for s in range(5):
        mask = (sp == s).astype(jnp.float32)
        acc = acc + jnp.dot(nf * mask, wskip_ref[s],
                            preferred_element_type=jnp.float32)
    self_ref[...] = acc


def _gather_body(h_hbm, snd_hbm, m_hbm, idx_v, rows_v,
                 isem0, isem1, gsem, wsem0, wsem1):
    c = lax.axis_index("c")
    s = lax.axis_index("s")
    wid = s * NC + c
    base = wid * G_CHUNKS
    isems = (isem0, isem1)
    wsems = (wsem0, wsem1)

    def issue_idx(g, slot):
        pltpu.async_copy(snd_hbm.at[base + g], idx_v.at[slot], isems[slot])

    def drain_idx(g, slot):
        pltpu.make_async_copy(snd_hbm.at[base + g], idx_v.at[slot],
                              isems[slot]).wait()

    def wb(g, slot):
        pltpu.async_copy(rows_v.at[slot],
                         m_hbm.at[pl.ds((base + g) * CHUNK, CHUNK)],
                         wsems[slot])

    def drain_wb(g, slot):
        pltpu.make_async_copy(rows_v.at[slot],
                              m_hbm.at[pl.ds((base + g) * CHUNK, CHUNK)],
                              wsems[slot]).wait()

    issue_idx(0, 0)
    issue_idx(1, 1)

    def pair_body(gp, carry):
        g0 = 2 * gp

        @pl.when(gp > 0)
        def _():
            drain_wb(g0 - 2, 0)
        drain_idx(g0, 0)
        pltpu.async_copy(h_hbm.at[idx_v.at[0]], rows_v.at[0], gsem).wait()
        issue_idx(g0 + 2, 0)
        wb(g0, 0)

        g1 = g0 + 1

        @pl.when(gp > 0)
        def _():
            drain_wb(g1 - 2, 1)
        drain_idx(g1, 1)
        pltpu.async_copy(h_hbm.at[idx_v.at[1]], rows_v.at[1], gsem).wait()

        @pl.when(g1 + 2 < G_CHUNKS)
        def _():
            issue_idx(g1 + 2, 1)
        wb(g1, 1)
        return carry

    lax.fori_loop(0, G_CHUNKS // 2, pair_body, 0)
    g = G_CHUNKS - 1
    drain_wb(g - 2, 0)
    drain_idx(g, 0)
    pltpu.async_copy(h_hbm.at[idx_v.at[0]], rows_v.at[0], gsem).wait()
    wb(g, 0)
    drain_wb(g - 1, 1)
    drain_wb(g, 0)


def _edge_kernel(vec_ref, m_ref, w1_ref, w2_ref, w3_ref,
                 wd0_ref, wd12_ref, s_ref, cy_ref, by_ref, ua_ref, ub_ref):
    v = vec_ref[...]
    x = v[:, 0:1]
    y = v[:, 1:2]
    z = v[:, 2:3]
    len2 = x * x + y * y + z * z
    zmask = len2 == 0.0
    inv_safe = jnp.where(zmask, 1.0, lax.rsqrt(len2))
    length = len2 * inv_safe

    u3 = v * inv_safe
    u3r = jnp.concatenate([u3[:, 1:3], u3[:, 0:1]], axis=1)
    p9 = jnp.concatenate([u3, u3 * u3, u3 * u3r], axis=1)
    yb = jnp.dot(p9, cy_ref[...],
                 preferred_element_type=jnp.float32) + by_ref[...]

    Be = v.shape[0]
    k = lax.broadcasted_iota(jnp.int32, (Be, NBASIS), 1).astype(jnp.float32) + 1.0
    kx = k * length
    n_i = (kx + 0.5).astype(jnp.int32)
    r = (kx - n_i.astype(jnp.float32)) * jnp.pi
    r2 = r * r
    poly = r * (1.0 + r2 * (-0.16666667 + r2 * (8.3333333e-3
                + r2 * (-1.98412698e-4 + r2 * 2.75573192e-6))))
    sgn = jnp.where((n_i & 1) == 0, 1.0, -1.0)
    bes = (1.4142135623730951 * sgn * poly) * inv_safe
    l6 = len2 * len2 * len2
    l7 = l6 * length
    l8 = l7 * length
    env = 1.0 - 28.0 * l6 + 48.0 * l7 - 21.0 * l8
    env = jnp.where(length < 1.0, env, 0.0)
    rad = jnp.where(zmask, 0.0, bes * env)

    hm = _swish(jnp.dot(rad, w1_ref[...], preferred_element_type=jnp.float32))
    hm = _swish(jnp.dot(hm, w2_ref[...], preferred_element_type=jnp.float32))
    mix = jnp.dot(hm, w3_ref[...], preferred_element_type=jnp.float32)

    m = m_ref[...]
    q0 = jnp.dot(m * mix[:, 0:D], wd0_ref[...],
                 preferred_element_type=jnp.float32)
    a12 = jnp.concatenate([m * mix[:, D:2 * D], m * mix[:, 2 * D:3 * D]],
                          axis=1)
    q12 = jnp.dot(a12, wd12_ref[...],
                  preferred_element_type=jnp.float32)
    qq = jnp.dot(q12, s_ref[...],
                 preferred_element_type=jnp.float32)
    u = jnp.concatenate([q0, qq * yb], axis=1)
    ua_ref[...] = u[:, :HALF]
    ub_ref[...] = u[:, HALF:]


def _scatter_body(ua_hbm, ub_hbm, recv_hbm, z_hbm, outa_hbm, outb_hbm,
                  idx_v, u_v, acc, sem_a, sem_b):
    c = lax.axis_index("c")
    sid = lax.axis_index("s")
    sems = (sem_a, sem_b)

    def run(u_hbm, out_hbm):
        pltpu.sync_copy(z_hbm, acc.at[pl.ds(sid * NODES_PER_TILE,
                                            NODES_PER_TILE)])
        plsc.subcore_barrier()
        base = sid * S_CHUNKS

        def issue(g, slot):
            r = base + g
            pltpu.async_copy(recv_hbm.at[r], idx_v.at[slot], sems[slot])
            pltpu.async_copy(u_hbm.at[pl.ds(r * CHUNK_S, CHUNK_S)],
                             u_v.at[slot], sems[slot])

        def drain(g, slot):
            r = base + g
            pltpu.make_async_copy(recv_hbm.at[r], idx_v.at[slot],
                                  sems[slot]).wait()
            pltpu.make_async_copy(u_hbm.at[pl.ds(r * CHUNK_S, CHUNK_S)],
                                  u_v.at[slot], sems[slot]).wait()

        def scat(slot):
            pltpu.sync_copy(u_v.at[slot], acc.at[idx_v.at[slot]], add=True)

        issue(0, 0)

        def pair_body(gp, carry):
            g0 = 2 * gp
            issue(g0 + 1, 1)
            drain(g0, 0)
            scat(0)

            @pl.when(g0 + 2 < S_CHUNKS)
            def _():
                issue(g0 + 2, 0)
            drain(g0 + 1, 1)
            scat(1)
            return carry

        lax.fori_loop(0, S_CHUNKS // 2, pair_body, 0)
        plsc.subcore_barrier()
        sl = pl.ds(sid * NODES_PER_TILE, NODES_PER_TILE)
        pltpu.sync_copy(acc.at[sl], out_hbm.at[sl])

    @pl.when(c == 0)
    def _():
        run(ua_hbm, outa_hbm)

    @pl.when(c == 1)
    def _():
        run(ub_hbm, outb_hbm)


def _out_kernel(aa_ref, ab_ref, self_ref, out_ref):
    inv = 1.0 / jnp.sqrt(AVG_NEIGH)
    aa = aa_ref[...]
    ab = ab_ref[...]
    s_pre = aa[:, :SCAL_OUT] * inv + self_ref[...]
    feat = s_pre[:, :MUL]
    gv = _swish(s_pre[:, MUL:2 * MUL])
    gt = _swish(s_pre[:, 2 * MUL:3 * MUL])
    acc_rest = jnp.concatenate([aa[:, SCAL_OUT:], ab], axis=1) * inv
    v_im = acc_rest[:, :96] * jnp.concatenate([gv, gv, gv], axis=1)
    t_im = acc_rest[:, 96:] * jnp.concatenate([gt, gt, gt, gt, gt], axis=1)
    out_ref[...] = jnp.concatenate([_swish(feat), v_im, t_im], axis=1)


def kernel(vectors, node_feats, node_specie, senders, receivers,
           W_skip, W_up, W_mlp1, W_mlp2, W_mlp3, W_down0, W_down1, W_down2):
    n = node_feats.shape[0]
    f32 = jnp.float32

    Bn = 1000
    h, self_conn = pl.pallas_call(
        _node_kernel,
        grid=(n // Bn,),
        in_specs=[
            pl.BlockSpec((Bn, D), lambda i: (i, 0)),
            pl.BlockSpec((Bn, 1), lambda i: (i, 0)),
            pl.BlockSpec((D, D), lambda i: (0, 0)),
            pl.BlockSpec((5, D, SCAL_OUT), lambda i: (0, 0, 0)),
        ],
        out_specs=[
            pl.BlockSpec((Bn, D), lambda i: (i, 0)),
            pl.BlockSpec((Bn, SCAL_OUT), lambda i: (i, 0)),
        ],
        out_shape=[
            jax.ShapeDtypeStruct((n, D), f32),
            jax.ShapeDtypeStruct((n, SCAL_OUT), f32),
        ],
    )(node_feats, node_specie.astype(jnp.int32).reshape(n, 1), W_up, W_skip)

    snd2d = senders.astype(jnp.int32).reshape(ROWS_2D, CHUNK)
    mesh = plsc.VectorSubcoreMesh(core_axis_name="c", subcore_axis_name="s")
    sc_params = pltpu.CompilerParams(use_tc_tiling_on_sc=False)
    m = pl.kernel(
        _gather_body,
        out_type=jax.ShapeDtypeStruct((N_EDGES, D), f32),
        mesh=mesh,
        compiler_params=sc_params,
        scratch_types=[
            pltpu.VMEM((2, CHUNK), jnp.int32),
            pltpu.VMEM((2, CHUNK, D), f32),
            pltpu.SemaphoreType.DMA,
            pltpu.SemaphoreType.DMA,
            pltpu.SemaphoreType.DMA,
            pltpu.SemaphoreType.DMA,
            pltpu.SemaphoreType.DMA,
        ],
    )(h, snd2d)

    s3 = 1.7320508075688772
    s15 = 3.872983346207417
    s5h = 1.118033988749895
    C9 = jnp.zeros((9, 8), f32)
    C9 = C9.at[0, 0].set(s3).at[1, 1].set(s3).at[2, 2].set(s3)
    C9 = C9.at[6, 3].set(s15)
    C9 = C9.at[7, 4].set(s15)
    C9 = C9.at[5, 5].set(3.0 * s5h)
    C9 = C9.at[8, 6].set(s15)
    C9 = C9.at[3, 7].set(0.5 * s15).at[4, 7].set(-0.5 * s15)
    b8 = jnp.zeros((1, 8), f32).at[0, 5].set(-s5h)
    R = jnp.kron(jnp.eye(8, dtype=f32), jnp.ones((1, MUL), f32))
    CY = C9 @ R
    bY = b8 @ R
    I32 = jnp.eye(MUL, dtype=f32)
    S = jnp.concatenate([
        jnp.concatenate([jnp.tile(I32, (1, 3)), jnp.zeros((MUL, 160), f32)], 1),
        jnp.concatenate([jnp.zeros((MUL, 96), f32), jnp.tile(I32, (1, 5))], 1),
    ], axis=0)
    Wd12 = jnp.concatenate([
        jnp.concatenate([W_down1, jnp.zeros((D, MUL), f32)], 1),
        jnp.concatenate([jnp.zeros((D, MUL), f32), W_down2], 1),
    ], axis=0)

    Be = 2000
    ua, ub = pl.pallas_call(
        _edge_kernel,
        grid=(N_EDGES // Be,),
        in_specs=[
            pl.BlockSpec((Be, 3), lambda i: (i, 0)),
            pl.BlockSpec((Be, D), lambda i: (i, 0)),
            pl.BlockSpec((NBASIS, HID), lambda i: (0, 0)),
            pl.BlockSpec((HID, HID), lambda i: (0, 0)),
            pl.BlockSpec((HID, 3 * D), lambda i: (0, 0)),
            pl.BlockSpec((D, SCAL_OUT), lambda i: (0, 0)),
            pl.BlockSpec((2 * D, HID), lambda i: (0, 0)),
            pl.BlockSpec((HID, 2 * D), lambda i: (0, 0)),
            pl.BlockSpec((9, 2 * D), lambda i: (0, 0)),
            pl.BlockSpec((1, 2 * D), lambda i: (0, 0)),
        ],
        out_specs=[
            pl.BlockSpec((Be, HALF), lambda i: (i, 0)),
            pl.BlockSpec((Be, HALF), lambda i: (i, 0)),
        ],
        out_shape=[
            jax.ShapeDtypeStruct((N_EDGES, HALF), f32),
            jax.ShapeDtypeStruct((N_EDGES, HALF), f32),
        ],
    )(vectors, m, W_mlp1, W_mlp2, W_mlp3, W_down0, Wd12, S, CY, bY)

    rcv2d = receivers.astype(jnp.int32).reshape(ROWS_S, CHUNK_S)
    zrows = jnp.zeros((NODES_PER_TILE, HALF), f32)
    acc_a, acc_b = pl.kernel(
        _scatter_body,
        out_type=[
            jax.ShapeDtypeStruct((N_PAD, HALF), f32),
            jax.ShapeDtypeStruct((N_PAD, HALF), f32),
        ],
        mesh=mesh,
        compiler_params=sc_params,
        scratch_types=[
            pltpu.VMEM((2, CHUNK_S), jnp.int32),
            pltpu.VMEM((2, CHUNK_S, HALF), f32),
            pltpu.VMEM_SHARED((N_PAD, HALF), f32),
            pltpu.SemaphoreType.DMA,
            pltpu.SemaphoreType.DMA,
        ],
    )(ua, ub, rcv2d, zrows)

    out288 = pl.pallas_call(
        _out_kernel,
        grid=(n // Bn,),
        in_specs=[
            pl.BlockSpec((Bn, HALF), lambda i: (i, 0)),
            pl.BlockSpec((Bn, HALF), lambda i: (i, 0)),
            pl.BlockSpec((Bn, SCAL_OUT), lambda i: (i, 0)),
        ],
        out_specs=pl.BlockSpec((Bn, 288), lambda i: (i, 0)),
        out_shape=jax.ShapeDtypeStruct((n, 288), f32),
    )(acc_a, acc_b, self_conn)

    out_s = out288[:, :MUL]
    out_v = out288[:, MUL:MUL + 96].reshape(n, 3, MUL).transpose(0, 2, 1)
    out_t = out288[:, MUL + 96:].reshape(n, 5, MUL).transpose(0, 2, 1)
    return jnp.concatenate(
        [out_s, out_v.reshape(n, 96), out_t.reshape(n, 160)], axis=1)


# speedup vs baseline: 31.9487x; 1.0081x over previous
import functools

import jax
import jax.numpy as jnp
from jax import lax
from jax.experimental import pallas as pl
from jax.experimental.pallas import tpu as pltpu
from jax.experimental.pallas import tpu_sc as plsc

N_NODES = 10000
N_EDGES = 320000
D = 128
MUL = 32
NBASIS = 8
HID = 64
SCAL_OUT = 96
AVG_NEIGH = 32.0

NC = 2
NS = 16
CHUNK = 80
SUB = 5
OC = SUB * CHUNK
OC_N = N_EDGES // (NC * NS) // OC
CHUNK_S = 40
ROWS_S = N_EDGES // CHUNK_S
S_CHUNKS = N_EDGES // NS // CHUNK_S
N_PAD = 10240
NODES_PER_TILE = N_PAD // NS
HALF = 176


def _swish(x):
    return x * (1.0 / (1.0 + jnp.exp(-x)))


def _node_kernel(nf_ref, sp_ref, wup_ref, wskip_ref, h_ref, self_ref):
    nf = nf_ref[...]
    sp = sp_ref[...]
    h_ref[...] = jnp.dot(nf, wup_ref[...], preferred_element_type=jnp.float32)
    acc = jnp.zeros((nf.shape[0], SCAL_OUT), jnp.float32)
    for s in range(5):
        mask = (sp == s).astype(jnp.float32)
        acc = acc + jnp.dot(nf * mask, wskip_ref[s],
                            preferred_element_type=jnp.float32)
    self_ref[...] = acc


def _gather_body(h_hbm, snd_hbm, m_hbm, idx_v, rows_v,
                 isem0, isem1, gsem, wsem0, wsem1):
    c = lax.axis_index("c")
    s = lax.axis_index("s")
    wid = s * NC + c
    base = wid * OC_N
    isems = (isem0, isem1)
    wsems = (wsem0, wsem1)

    def issue_idx(g, slot):
        pltpu.async_copy(snd_hbm.at[base + g], idx_v.at[slot], isems[slot])

    def drain_idx(g, slot):
        pltpu.make_async_copy(snd_hbm.at[base + g], idx_v.at[slot],
                              isems[slot]).wait()

    def gather(slot):
        for j in range(SUB):
            pltpu.async_copy(h_hbm.at[idx_v.at[slot, j]],
                             rows_v.at[slot, pl.ds(j * CHUNK, CHUNK)], gsem)
        for j in range(SUB):
            pltpu.make_async_copy(h_hbm.at[idx_v.at[slot, j]],
                                  rows_v.at[slot, pl.ds(j * CHUNK, CHUNK)],
                                  gsem).wait()

    def wb(g, slot):
        pltpu.async_copy(rows_v.at[slot],
                         m_hbm.at[pl.ds((base + g) * OC, OC)], wsems[slot])

    def drain_wb(g, slot):
        pltpu.make_async_copy(rows_v.at[slot],
                              m_hbm.at[pl.ds((base + g) * OC, OC)],
                              wsems[slot]).wait()

    issue_idx(0, 0)
    issue_idx(1, 1)

    def pair_body(gp, carry):
        g0 = 2 * gp

        @pl.when(gp > 0)
        def _():
            drain_wb(g0 - 2, 0)
        drain_idx(g0, 0)
        gather(0)
        issue_idx(g0 + 2, 0)
        wb(g0, 0)

        g1 = g0 + 1

        @pl.when(gp > 0)
        def _():
            drain_wb(g1 - 2, 1)
        drain_idx(g1, 1)
        gather(1)

        @pl.when(g1 + 2 < OC_N)
        def _():
            issue_idx(g1 + 2, 1)
        wb(g1, 1)
        return carry

    lax.fori_loop(0, OC_N // 2, pair_body, 0)
    g = OC_N - 1
    drain_wb(g - 2, 0)
    drain_idx(g, 0)
    gather(0)
    wb(g, 0)
    drain_wb(g - 1, 1)
    drain_wb(g, 0)


def _edge_kernel(vec_ref, m_ref, w1_ref, w2_ref, w3_ref,
                 wd0_ref, wd12_ref, s_ref, cy_ref, by_ref, ua_ref, ub_ref):
    v = vec_ref[...]
    x = v[:, 0:1]
    y = v[:, 1:2]
    z = v[:, 2:3]
    len2 = x * x + y * y + z * z
    zmask = len2 == 0.0
    inv_safe = jnp.where(zmask, 1.0, lax.rsqrt(len2))
    length = len2 * inv_safe

    u3 = v * inv_safe
    u3r = jnp.concatenate([u3[:, 1:3], u3[:, 0:1]], axis=1)
    p9 = jnp.concatenate([u3, u3 * u3, u3 * u3r], axis=1)
    yb = jnp.dot(p9, cy_ref[...],
                 preferred_element_type=jnp.float32) + by_ref[...]

    Be = v.shape[0]
    k = lax.broadcasted_iota(jnp.int32, (Be, NBASIS), 1).astype(jnp.float32) + 1.0
    kx = k * length
    n_i = (kx + 0.5).astype(jnp.int32)
    r = (kx - n_i.astype(jnp.float32)) * jnp.pi
    r2 = r * r
    poly = r * (1.0 + r2 * (-0.16666667 + r2 * (8.3333333e-3
                + r2 * (-1.98412698e-4 + r2 * 2.75573192e-6))))
    sgn = jnp.where((n_i & 1) == 0, 1.0, -1.0)
    bes = (1.4142135623730951 * sgn * poly) * inv_safe
    l6 = len2 * len2 * len2
    l7 = l6 * length
    l8 = l7 * length
    env = 1.0 - 28.0 * l6 + 48.0 * l7 - 21.0 * l8
    env = jnp.where(length < 1.0, env, 0.0)
    rad = jnp.where(zmask, 0.0, bes * env)

    hm = _swish(jnp.dot(rad, w1_ref[...], preferred_element_type=jnp.float32))
    hm = _swish(jnp.dot(hm, w2_ref[...], preferred_element_type=jnp.float32))
    mix = jnp.dot(hm, w3_ref[...], preferred_element_type=jnp.float32)

    m = m_ref[...]
    q0 = jnp.dot(m * mix[:, 0:D], wd0_ref[...],
                 preferred_element_type=jnp.float32)
    a12 = jnp.concatenate([m * mix[:, D:2 * D], m * mix[:, 2 * D:3 * D]],
                          axis=1)
    q12 = jnp.dot(a12, wd12_ref[...],
                  preferred_element_type=jnp.float32)
    qq = jnp.dot(q12, s_ref[...],
                 preferred_element_type=jnp.float32)
    u = jnp.concatenate([q0, qq * yb], axis=1)
    ua_ref[...] = u[:, :HALF]
    ub_ref[...] = u[:, HALF:]


def _scatter_body(ua_hbm, ub_hbm, recv_hbm, z_hbm, outa_hbm, outb_hbm,
                  idx_v, u_v, acc, sem_a, sem_b):
    c = lax.axis_index("c")
    sid = lax.axis_index("s")
    sems = (sem_a, sem_b)

    def run(u_hbm, out_hbm):
        pltpu.sync_copy(z_hbm, acc.at[pl.ds(sid * NODES_PER_TILE,
                                            NODES_PER_TILE)])
        plsc.subcore_barrier()
        base = sid * S_CHUNKS

        def issue(g, slot):
            r = base + g
            pltpu.async_copy(recv_hbm.at[r], idx_v.at[slot], sems[slot])
            pltpu.async_copy(u_hbm.at[pl.ds(r * CHUNK_S, CHUNK_S)],
                             u_v.at[slot], sems[slot])

        def drain(g, slot):
            r = base + g
            pltpu.make_async_copy(recv_hbm.at[r], idx_v.at[slot],
                                  sems[slot]).wait()
            pltpu.make_async_copy(u_hbm.at[pl.ds(r * CHUNK_S, CHUNK_S)],
                                  u_v.at[slot], sems[slot]).wait()

        def scat(slot):
            pltpu.sync_copy(u_v.at[slot], acc.at[idx_v.at[slot]], add=True)

        issue(0, 0)

        def pair_body(gp, carry):
            g0 = 2 * gp
            issue(g0 + 1, 1)
            drain(g0, 0)
            scat(0)

            @pl.when(g0 + 2 < S_CHUNKS)
            def _():
                issue(g0 + 2, 0)
            drain(g0 + 1, 1)
            scat(1)
            return carry

        lax.fori_loop(0, S_CHUNKS // 2, pair_body, 0)
        plsc.subcore_barrier()
        sl = pl.ds(sid * NODES_PER_TILE, NODES_PER_TILE)
        pltpu.sync_copy(acc.at[sl], out_hbm.at[sl])

    @pl.when(c == 0)
    def _():
        run(ua_hbm, outa_hbm)

    @pl.when(c == 1)
    def _():
        run(ub_hbm, outb_hbm)


def _out_kernel(aa_ref, ab_ref, self_ref, out_ref):
    inv = 1.0 / jnp.sqrt(AVG_NEIGH)
    aa = aa_ref[...]
    ab = ab_ref[...]
    s_pre = aa[:, :SCAL_OUT] * inv + self_ref[...]
    feat = s_pre[:, :MUL]
    gv = _swish(s_pre[:, MUL:2 * MUL])
    gt = _swish(s_pre[:, 2 * MUL:3 * MUL])
    acc_rest = jnp.concatenate([aa[:, SCAL_OUT:], ab], axis=1) * inv
    v_im = acc_rest[:, :96] * jnp.concatenate([gv, gv, gv], axis=1)
    t_im = acc_rest[:, 96:] * jnp.concatenate([gt, gt, gt, gt, gt], axis=1)
    out_ref[...] = jnp.concatenate([_swish(feat), v_im, t_im], axis=1)


def kernel(vectors, node_feats, node_specie, senders, receivers,
           W_skip, W_up, W_mlp1, W_mlp2, W_mlp3, W_down0, W_down1, W_down2):
    n = node_feats.shape[0]
    f32 = jnp.float32

    Bn = 1000
    h, self_conn = pl.pallas_call(
        _node_kernel,
        grid=(n // Bn,),
        in_specs=[
            pl.BlockSpec((Bn, D), lambda i: (i, 0)),
            pl.BlockSpec((Bn, 1), lambda i: (i, 0)),
            pl.BlockSpec((D, D), lambda i: (0, 0)),
            pl.BlockSpec((5, D, SCAL_OUT), lambda i: (0, 0, 0)),
        ],
        out_specs=[
            pl.BlockSpec((Bn, D), lambda i: (i, 0)),
            pl.BlockSpec((Bn, SCAL_OUT), lambda i: (i, 0)),
        ],
        out_shape=[
            jax.ShapeDtypeStruct((n, D), f32),
            jax.ShapeDtypeStruct((n, SCAL_OUT), f32),
        ],
    )(node_feats, node_specie.astype(jnp.int32).reshape(n, 1), W_up, W_skip)

    snd3d = senders.astype(jnp.int32).reshape(N_EDGES // OC, SUB, CHUNK)
    mesh = plsc.VectorSubcoreMesh(core_axis_name="c", subcore_axis_name="s")
    sc_params = pltpu.CompilerParams(use_tc_tiling_on_sc=False)
    m = pl.kernel(
        _gather_body,
        out_type=jax.ShapeDtypeStruct((N_EDGES, D), f32),
        mesh=mesh,
        scratch_types=[
            pltpu.VMEM((2, SUB, CHUNK), jnp.int32),
            pltpu.VMEM((2, OC, D), f32),
            pltpu.SemaphoreType.DMA,
            pltpu.SemaphoreType.DMA,
            pltpu.SemaphoreType.DMA,
            pltpu.SemaphoreType.DMA,
            pltpu.SemaphoreType.DMA,
        ],
    )(h, snd3d)

    s3 = 1.7320508075688772
    s15 = 3.872983346207417
    s5h = 1.118033988749895
    C9 = jnp.zeros((9, 8), f32)
    C9 = C9.at[0, 0].set(s3).at[1, 1].set(s3).at[2, 2].set(s3)
    C9 = C9.at[6, 3].set(s15)
    C9 = C9.at[7, 4].set(s15)
    C9 = C9.at[5, 5].set(3.0 * s5h)
    C9 = C9.at[8, 6].set(s15)
    C9 = C9.at[3, 7].set(0.5 * s15).at[4, 7].set(-0.5 * s15)
    b8 = jnp.zeros((1, 8), f32).at[0, 5].set(-s5h)
    R = jnp.kron(jnp.eye(8, dtype=f32), jnp.ones((1, MUL), f32))
    CY = C9 @ R
    bY = b8 @ R
    I32 = jnp.eye(MUL, dtype=f32)
    S = jnp.concatenate([
        jnp.concatenate([jnp.tile(I32, (1, 3)), jnp.zeros((MUL, 160), f32)], 1),
        jnp.concatenate([jnp.zeros((MUL, 96), f32), jnp.tile(I32, (1, 5))], 1),
    ], axis=0)
    Wd12 = jnp.concatenate([
        jnp.concatenate([W_down1, jnp.zeros((D, MUL), f32)], 1),
        jnp.concatenate([jnp.zeros((D, MUL), f32), W_down2], 1),
    ], axis=0)

    Be = 2000
    ua, ub = pl.pallas_call(
        _edge_kernel,
        grid=(N_EDGES // Be,),
        in_specs=[
            pl.BlockSpec((Be, 3), lambda i: (i, 0)),
            pl.BlockSpec((Be, D), lambda i: (i, 0)),
            pl.BlockSpec((NBASIS, HID), lambda i: (0, 0)),
            pl.BlockSpec((HID, HID), lambda i: (0, 0)),
            pl.BlockSpec((HID, 3 * D), lambda i: (0, 0)),
            pl.BlockSpec((D, SCAL_OUT), lambda i: (0, 0)),
            pl.BlockSpec((2 * D, HID), lambda i: (0, 0)),
            pl.BlockSpec((HID, 2 * D), lambda i: (0, 0)),
            pl.BlockSpec((9, 2 * D), lambda i: (0, 0)),
            pl.BlockSpec((1, 2 * D), lambda i: (0, 0)),
        ],
        out_specs=[
            pl.BlockSpec((Be, HALF), lambda i: (i, 0)),
            pl.BlockSpec((Be, HALF), lambda i: (i, 0)),
        ],
        out_shape=[
            jax.ShapeDtypeStruct((N_EDGES, HALF), f32),
            jax.ShapeDtypeStruct((N_EDGES, HALF), f32),
        ],
    )(vectors, m, W_mlp1, W_mlp2, W_mlp3, W_down0, Wd12, S, CY, bY)

    rcv2d = receivers.astype(jnp.int32).reshape(ROWS_S, CHUNK_S)
    zrows = jnp.zeros((NODES_PER_TILE, HALF), f32)
    acc_a, acc_b = pl.kernel(
        _scatter_body,
        out_type=[
            jax.ShapeDtypeStruct((N_PAD, HALF), f32),
            jax.ShapeDtypeStruct((N_PAD, HALF), f32),
        ],
        mesh=mesh,
        compiler_params=sc_params,
        scratch_types=[
            pltpu.VMEM((2, CHUNK_S), jnp.int32),
            pltpu.VMEM((2, CHUNK_S, HALF), f32),
            pltpu.VMEM_SHARED((N_PAD, HALF), f32),
            pltpu.SemaphoreType.DMA,
            pltpu.SemaphoreType.DMA,
        ],
    )(ua, ub, rcv2d, zrows)

    out288 = pl.pallas_call(
        _out_kernel,
        grid=(n // Bn,),
        in_specs=[
            pl.BlockSpec((Bn, HALF), lambda i: (i, 0)),
            pl.BlockSpec((Bn, HALF), lambda i: (i, 0)),
            pl.BlockSpec((Bn, SCAL_OUT), lambda i: (i, 0)),
        ],
        out_specs=pl.BlockSpec((Bn, 288), lambda i: (i, 0)),
        out_shape=jax.ShapeDtypeStruct((n, 288), f32),
    )(acc_a, acc_b, self_conn)

    out_s = out288[:, :MUL]
    out_v = out288[:, MUL:MUL + 96].reshape(n, 3, MUL).transpose(0, 2, 1)
    out_t = out288[:, MUL + 96:].reshape(n, 5, MUL).transpose(0, 2, 1)
    return jnp.concatenate(
        [out_s, out_v.reshape(n, 96), out_t.reshape(n, 160)], axis=1)
